# baseline (speedup 1.0000x reference)
import jax
import jax.numpy as jnp
from jax.experimental import pallas as pl

N = 10000
NUM_GRAPHS = 256
DIM = 128


def _gatv2_layer(h, src, dst, Wl, Wr, att, bias):
    xl = h @ Wl
    xr = h @ Wr
    e = jax.nn.leaky_relu(xl[src] + xr[dst], negative_slope=0.2) @ att
    m = jax.ops.segment_max(e, dst, num_segments=N)
    ex = jnp.exp(e - m[dst])
    denom = jax.ops.segment_sum(ex, dst, num_segments=N)
    alpha = ex / (denom[dst] + 1e-16)
    out = jax.ops.segment_sum(alpha[:, None] * xl[src], dst, num_segments=N)
    return out + bias


def _tail_kernel(pooled_ref, w_ref, b_ref, out_ref):
    out_ref[...] = pooled_ref[...] @ w_ref[...] + b_ref[0, 0]


def kernel(x, edge_index, batch, Wl1, Wr1, att1, b1, Wln, Wrn, attn, bn, Wout, bout):
    src = edge_index[0].astype(jnp.int32)
    dst = edge_index[1].astype(jnp.int32)
    h = _gatv2_layer(x, src, dst, Wl1, Wr1, att1, b1)
    h = jax.nn.relu(h)
    for _ in range(2):
        h = _gatv2_layer(h, src, dst, Wln, Wrn, attn, bn)
        h = jax.nn.relu(h)
    pooled = jax.ops.segment_sum(h, batch.astype(jnp.int32), num_segments=NUM_GRAPHS)
    out = pl.pallas_call(
        _tail_kernel,
        out_shape=jax.ShapeDtypeStruct((NUM_GRAPHS, 1), jnp.float32),
    )(pooled, Wout, bout.reshape(1, 1))
    return out


# speedup vs baseline: 7.1870x; 7.1870x over previous
import functools

import jax
import jax.numpy as jnp
from jax import lax
from jax.experimental import pallas as pl
from jax.experimental.pallas import tpu as pltpu
from jax.experimental.pallas import tpu_sc as plsc

N = 10000
E = 320000
DIM = 128
NUM_GRAPHS = 256

NC = 2
NS = 16
NW = NC * NS
E_TILE = E // NW
CHUNK = 80
NG = CHUNK // 16
NCHUNK = E_TILE // CHUNK
ROW_TILE = 624
R8 = 8

NBLK = 1000
NGRID = N // NBLK


def _row_span(s):
    row_lo = s * ROW_TILE
    n8 = jnp.where(s == NS - 1, (N - (NS - 1) * ROW_TILE) // R8, ROW_TILE // R8)
    return row_lo, n8


def _sc_numerator(xl, xr, att, src, dst):
    mesh = plsc.VectorSubcoreMesh(core_axis_name="c", subcore_axis_name="s")

    @functools.partial(
        pl.kernel,
        out_type=(
            jax.ShapeDtypeStruct((NC, N, DIM), jnp.float32),
            jax.ShapeDtypeStruct((E,), jnp.float32),
        ),
        mesh=mesh,
        scratch_types=(
            pltpu.VMEM_SHARED((N, DIM), jnp.float32),
            pltpu.VMEM((CHUNK,), jnp.int32),
            pltpu.VMEM((CHUNK,), jnp.int32),
            pltpu.VMEM((CHUNK, DIM), jnp.float32),
            pltpu.VMEM((CHUNK, DIM), jnp.float32),
            pltpu.VMEM((CHUNK,), jnp.float32),
            pltpu.VMEM((DIM,), jnp.float32),
            pltpu.VMEM((R8, DIM), jnp.float32),
        ),
    )
    def body(xl_h, xr_h, att_h, src_h, dst_h, out_n, out_w,
             accn, srcv, dstv, xlr, xrr, wv, attv, zbn):
        c = lax.axis_index("c")
        s = lax.axis_index("s")
        tid = c * NS + s
        zro = jnp.zeros((16,), jnp.float32)

        def zrow(r, _):
            for k in range(DIM // 16):
                zbn[r, pl.ds(k * 16, 16)] = zro
            return 0
        lax.fori_loop(0, R8, zrow, 0)

        row_lo, n8 = _row_span(s)

        def zacc(r, _):
            pltpu.sync_copy(zbn, accn.at[pl.ds(row_lo + r * R8, R8)])
            return 0
        lax.fori_loop(0, n8, zacc, 0)
        plsc.subcore_barrier()

        pltpu.sync_copy(att_h, attv)
        attc = [attv[pl.ds(k * 16, 16)] for k in range(DIM // 16)]
        lane_i = lax.broadcasted_iota(jnp.int32, (16,), 0)
        perms = [jnp.bitwise_xor(lane_i, sh) for sh in (1, 2, 4, 8)]

        def hsum(v):
            for pm in perms:
                v = v + v.at[pm].get(mode="promise_in_bounds")
            return v

        def chunk_body(i, _):
            base = tid * E_TILE + i * CHUNK
            pltpu.sync_copy(src_h.at[pl.ds(base, CHUNK)], srcv)
            pltpu.sync_copy(dst_h.at[pl.ds(base, CHUNK)], dstv)
            pltpu.sync_copy(xl_h.at[srcv], xlr)
            pltpu.sync_copy(xr_h.at[dstv], xrr)

            def group_body(g, _):
                def p1(e16, evec):
                    e = g * 16 + e16
                    p = zro
                    for k in range(DIM // 16):
                        sl = pl.ds(k * 16, 16)
                        z = xlr[e, sl] + xrr[e, sl]
                        y = jnp.maximum(z, 0.2 * z)
                        p = p + y * attc[k]
                    return jnp.where(lane_i == e16, hsum(p), evec)
                evec = lax.fori_loop(0, 16, p1, zro)
                exvec = jnp.exp(evec)
                wv[pl.ds(g * 16, 16)] = exvec

                def p2(e16, _):
                    e = g * 16 + e16
                    w = exvec.at[jnp.full((16,), e16, jnp.int32)].get(
                        mode="promise_in_bounds")
                    for k in range(DIM // 16):
                        sl = pl.ds(k * 16, 16)
                        xlr[e, sl] = w * xlr[e, sl]
                    return 0
                lax.fori_loop(0, 16, p2, 0)
                return 0
            lax.fori_loop(0, NG, group_body, 0)

            pltpu.sync_copy(wv, out_w.at[pl.ds(base, CHUNK)])
            pltpu.sync_copy(xlr, accn.at[dstv], add=True)
            return 0
        lax.fori_loop(0, NCHUNK, chunk_body, 0)

        plsc.subcore_barrier()

        def wout(r, _):
            off = row_lo + r * R8
            pltpu.sync_copy(accn.at[pl.ds(off, R8)],
                            out_n.at[c, pl.ds(off, R8)])
            return 0
        lax.fori_loop(0, n8, wout, 0)

    return body(xl, xr, att, src, dst)


def _sc_denominator(w_edge, dst):
    mesh = plsc.VectorSubcoreMesh(core_axis_name="c", subcore_axis_name="s")

    @functools.partial(
        pl.kernel,
        out_type=jax.ShapeDtypeStruct((NC, N, 16), jnp.float32),
        mesh=mesh,
        scratch_types=(
            pltpu.VMEM_SHARED((N, 16), jnp.float32),
            pltpu.VMEM((CHUNK,), jnp.int32),
            pltpu.VMEM((CHUNK,), jnp.float32),
            pltpu.VMEM((CHUNK, 16), jnp.float32),
            pltpu.VMEM((R8, 16), jnp.float32),
        ),
    )
    def body(w_h, dst_h, out_d, accd, dstv, wv, den, zbd):
        c = lax.axis_index("c")
        s = lax.axis_index("s")
        tid = c * NS + s
        zro = jnp.zeros((16,), jnp.float32)
        lane_i = lax.broadcasted_iota(jnp.int32, (16,), 0)

        def zrow(r, _):
            zbd[r, pl.ds(0, 16)] = zro
            return 0
        lax.fori_loop(0, R8, zrow, 0)

        row_lo, n8 = _row_span(s)

        def zacc(r, _):
            pltpu.sync_copy(zbd, accd.at[pl.ds(row_lo + r * R8, R8)])
            return 0
        lax.fori_loop(0, n8, zacc, 0)
        plsc.subcore_barrier()

        def chunk_body(i, _):
            base = tid * E_TILE + i * CHUNK
            pltpu.sync_copy(dst_h.at[pl.ds(base, CHUNK)], dstv)
            pltpu.sync_copy(w_h.at[pl.ds(base, CHUNK)], wv)

            def group_body(g, _):
                exvec = wv[pl.ds(g * 16, 16)]

                def p2(e16, _):
                    e = g * 16 + e16
                    w = exvec.at[jnp.full((16,), e16, jnp.int32)].get(
                        mode="promise_in_bounds")
                    den[e, pl.ds(0, 16)] = jnp.where(lane_i == e16, w, 0.0)
                    return 0
                lax.fori_loop(0, 16, p2, 0)
                return 0
            lax.fori_loop(0, NG, group_body, 0)

            pltpu.sync_copy(den, accd.at[dstv], add=True)
            return 0
        lax.fori_loop(0, NCHUNK, chunk_body, 0)

        plsc.subcore_barrier()

        def wout(r, _):
            off = row_lo + r * R8
            pltpu.sync_copy(accd.at[pl.ds(off, R8)],
                            out_d.at[c, pl.ds(off, R8)])
            return 0
        lax.fori_loop(0, n8, wout, 0)

    return body(w_edge, dst)


def _sc_edge_attention(xl, xr, att, src, dst):
    num, w_edge = _sc_numerator(xl, xr, att, src, dst)
    den = _sc_denominator(w_edge, dst)
    return num, den


def _tc_first_kernel(x_ref, wl_ref, wr_ref, xl_ref, xr_ref):
    x = x_ref[...]
    xl_ref[...] = jnp.dot(x, wl_ref[...], preferred_element_type=jnp.float32)
    xr_ref[...] = jnp.dot(x, wr_ref[...], preferred_element_type=jnp.float32)


def _tc_first(x, wl, wr):
    return pl.pallas_call(
        _tc_first_kernel,
        grid=(NGRID,),
        in_specs=[
            pl.BlockSpec((NBLK, DIM), lambda i: (i, 0)),
            pl.BlockSpec((DIM, DIM), lambda i: (0, 0)),
            pl.BlockSpec((DIM, DIM), lambda i: (0, 0)),
        ],
        out_specs=[
            pl.BlockSpec((NBLK, DIM), lambda i: (i, 0)),
            pl.BlockSpec((NBLK, DIM), lambda i: (i, 0)),
        ],
        out_shape=[
            jax.ShapeDtypeStruct((N, DIM), jnp.float32),
            jax.ShapeDtypeStruct((N, DIM), jnp.float32),
        ],
    )(x, wl, wr)


def _tc_mid_kernel(n_ref, d_ref, b_ref, wl_ref, wr_ref, xl_ref, xr_ref):
    num = n_ref[0] + n_ref[1]
    den = jnp.sum(d_ref[0] + d_ref[1], axis=1, keepdims=True)
    h = jnp.maximum(num / (den + 1e-16) + b_ref[0], 0.0)
    xl_ref[...] = jnp.dot(h, wl_ref[...], preferred_element_type=jnp.float32)
    xr_ref[...] = jnp.dot(h, wr_ref[...], preferred_element_type=jnp.float32)


def _tc_mid(num, den, b, wl, wr):
    return pl.pallas_call(
        _tc_mid_kernel,
        grid=(NGRID,),
        in_specs=[
            pl.BlockSpec((NC, NBLK, DIM), lambda i: (0, i, 0)),
            pl.BlockSpec((NC, NBLK, 16), lambda i: (0, i, 0)),
            pl.BlockSpec((1, DIM), lambda i: (0, 0)),
            pl.BlockSpec((DIM, DIM), lambda i: (0, 0)),
            pl.BlockSpec((DIM, DIM), lambda i: (0, 0)),
        ],
        out_specs=[
            pl.BlockSpec((NBLK, DIM), lambda i: (i, 0)),
            pl.BlockSpec((NBLK, DIM), lambda i: (i, 0)),
        ],
        out_shape=[
            jax.ShapeDtypeStruct((N, DIM), jnp.float32),
            jax.ShapeDtypeStruct((N, DIM), jnp.float32),
        ],
    )(num, den, b.reshape(1, DIM), wl, wr)


def _tc_final_kernel(n_ref, d_ref, b_ref, batch_ref, wout_ref, bout_ref, out_ref):
    i = pl.program_id(0)
    num = n_ref[0] + n_ref[1]
    den = jnp.sum(d_ref[0] + d_ref[1], axis=1, keepdims=True)
    h = jnp.maximum(num / (den + 1e-16) + b_ref[0], 0.0)
    v = jnp.dot(h, wout_ref[...], preferred_element_type=jnp.float32)
    bb = batch_ref[0, 0, :]
    gids = lax.broadcasted_iota(jnp.int32, (NUM_GRAPHS, NBLK), 0)
    oh = (bb[None, :] == gids).astype(jnp.float32)
    contrib = jnp.dot(oh, v, preferred_element_type=jnp.float32)

    @pl.when(i == 0)
    def _():
        out_ref[...] = contrib + bout_ref[0, 0]

    @pl.when(i > 0)
    def _():
        out_ref[...] = out_ref[...] + contrib


def _tc_final(num, den, b, batch3, wout, bout):
    return pl.pallas_call(
        _tc_final_kernel,
        grid=(NGRID,),
        in_specs=[
            pl.BlockSpec((NC, NBLK, DIM), lambda i: (0, i, 0)),
            pl.BlockSpec((NC, NBLK, 16), lambda i: (0, i, 0)),
            pl.BlockSpec((1, DIM), lambda i: (0, 0)),
            pl.BlockSpec((1, 1, NBLK), lambda i: (i, 0, 0)),
            pl.BlockSpec((DIM, 1), lambda i: (0, 0)),
            pl.BlockSpec((1, 1), lambda i: (0, 0)),
        ],
        out_specs=pl.BlockSpec((NUM_GRAPHS, 1), lambda i: (0, 0)),
        out_shape=jax.ShapeDtypeStruct((NUM_GRAPHS, 1), jnp.float32),
    )(num, den, b.reshape(1, DIM), batch3, wout, bout.reshape(1, 1))


def kernel(x, edge_index, batch, Wl1, Wr1, att1, b1, Wln, Wrn, attn, bn, Wout, bout):
    src = edge_index[0].astype(jnp.int32)
    dst = edge_index[1].astype(jnp.int32)
    batch3 = batch.astype(jnp.int32).reshape(NGRID, 1, NBLK)

    xl, xr = _tc_first(x, Wl1, Wr1)
    num, den = _sc_edge_attention(xl, xr, att1, src, dst)
    xl, xr = _tc_mid(num, den, b1, Wln, Wrn)
    num, den = _sc_edge_attention(xl, xr, attn, src, dst)
    xl, xr = _tc_mid(num, den, bn, Wln, Wrn)
    num, den = _sc_edge_attention(xl, xr, attn, src, dst)
    return _tc_final(num, den, bn, batch3, Wout, bout)


# speedup vs baseline: 8.0969x; 1.1266x over previous
import functools

import jax
import jax.numpy as jnp
from jax import lax
from jax.experimental import pallas as pl
from jax.experimental.pallas import tpu as pltpu
from jax.experimental.pallas import tpu_sc as plsc

N = 10000
E = 320000
DIM = 128
NUM_GRAPHS = 256

NC = 2
NS = 16
NW = NC * NS
E_TILE = E // NW
CHUNK = 80
NG = CHUNK // 16
NCHUNK = E_TILE // CHUNK
ROW_TILE = 624
R8 = 8

NBLK = 1000
NGRID = N // NBLK


def _row_span(s):
    row_lo = s * ROW_TILE
    n8 = jnp.where(s == NS - 1, (N - (NS - 1) * ROW_TILE) // R8, ROW_TILE // R8)
    return row_lo, n8


def _sc_numerator(xl, xr, att, src, dst):
    mesh = plsc.VectorSubcoreMesh(core_axis_name="c", subcore_axis_name="s")

    @functools.partial(
        pl.kernel,
        out_type=(
            jax.ShapeDtypeStruct((NC, N, DIM), jnp.float32),
            jax.ShapeDtypeStruct((E,), jnp.float32),
        ),
        mesh=mesh,
        scratch_types=(
            pltpu.VMEM_SHARED((N, DIM), jnp.float32),
            pltpu.VMEM((CHUNK,), jnp.int32),
            pltpu.VMEM((CHUNK,), jnp.int32),
            pltpu.VMEM((CHUNK, DIM), jnp.float32),
            pltpu.VMEM((CHUNK, DIM), jnp.float32),
            pltpu.VMEM((CHUNK,), jnp.float32),
            pltpu.VMEM((DIM,), jnp.float32),
            pltpu.VMEM((R8, DIM), jnp.float32),
        ),
    )
    def body(xl_h, xr_h, att_h, src_h, dst_h, out_n, out_w,
             accn, srcv, dstv, xlr, xrr, wv, attv, zbn):
        c = lax.axis_index("c")
        s = lax.axis_index("s")
        tid = c * NS + s
        zro = jnp.zeros((16,), jnp.float32)

        def zrow(r, _):
            for k in range(DIM // 16):
                zbn[r, pl.ds(k * 16, 16)] = zro
            return 0
        lax.fori_loop(0, R8, zrow, 0)

        row_lo, n8 = _row_span(s)

        def zacc(r, _):
            pltpu.sync_copy(zbn, accn.at[pl.ds(row_lo + r * R8, R8)])
            return 0
        lax.fori_loop(0, n8, zacc, 0)
        plsc.subcore_barrier()

        pltpu.sync_copy(att_h, attv)
        attc = [attv[pl.ds(k * 16, 16)] for k in range(DIM // 16)]
        lane_i = lax.broadcasted_iota(jnp.int32, (16,), 0)
        perms = [jnp.bitwise_xor(lane_i, sh) for sh in (1, 2, 4, 8)]

        def hsum(v):
            for pm in perms:
                v = v + v.at[pm].get(mode="promise_in_bounds")
            return v

        def chunk_body(i, _):
            base = tid * E_TILE + i * CHUNK
            pltpu.sync_copy(src_h.at[pl.ds(base, CHUNK)], srcv)
            pltpu.sync_copy(dst_h.at[pl.ds(base, CHUNK)], dstv)
            pltpu.sync_copy(xl_h.at[srcv], xlr)
            pltpu.sync_copy(xr_h.at[dstv], xrr)

            def group_body(g, _):
                def p1(e16, evec):
                    e = g * 16 + e16
                    p = zro
                    for k in range(DIM // 16):
                        sl = pl.ds(k * 16, 16)
                        z = xlr[e, sl] + xrr[e, sl]
                        y = jnp.maximum(z, 0.2 * z)
                        p = p + y * attc[k]
                    return jnp.where(lane_i == e16, hsum(p), evec)
                evec = lax.fori_loop(0, 16, p1, zro)
                exvec = jnp.exp(evec)
                wv[pl.ds(g * 16, 16)] = exvec

                def p2(e16, _):
                    e = g * 16 + e16
                    w = exvec.at[jnp.full((16,), e16, jnp.int32)].get(
                        mode="promise_in_bounds")
                    for k in range(DIM // 16):
                        sl = pl.ds(k * 16, 16)
                        xlr[e, sl] = w * xlr[e, sl]
                    return 0
                lax.fori_loop(0, 16, p2, 0)
                return 0
            lax.fori_loop(0, NG, group_body, 0)

            pltpu.sync_copy(wv, out_w.at[pl.ds(base, CHUNK)])
            pltpu.sync_copy(xlr, accn.at[dstv], add=True)
            return 0
        lax.fori_loop(0, NCHUNK, chunk_body, 0)

        plsc.subcore_barrier()

        def wout(r, _):
            off = row_lo + r * R8
            pltpu.sync_copy(accn.at[pl.ds(off, R8)],
                            out_n.at[c, pl.ds(off, R8)])
            return 0
        lax.fori_loop(0, n8, wout, 0)

    return body(xl, xr, att, src, dst)


def _sc_denominator(w_edge, dst):
    mesh = plsc.VectorSubcoreMesh(core_axis_name="c", subcore_axis_name="s")

    @functools.partial(
        pl.kernel,
        out_type=jax.ShapeDtypeStruct((NC, N, 16), jnp.float32),
        mesh=mesh,
        scratch_types=(
            pltpu.VMEM_SHARED((N, 16), jnp.float32),
            pltpu.VMEM((NCHUNK, CHUNK), jnp.int32),
            pltpu.VMEM((E_TILE,), jnp.float32),
            pltpu.VMEM((CHUNK, 16), jnp.float32),
            pltpu.VMEM((R8, 16), jnp.float32),
        ),
    )
    def body(w_h, dst_h, out_d, accd, dstall, wall, den, zbd):
        c = lax.axis_index("c")
        s = lax.axis_index("s")
        tid = c * NS + s
        zro = jnp.zeros((16,), jnp.float32)
        lane_i = lax.broadcasted_iota(jnp.int32, (16,), 0)

        def zrow(r, _):
            zbd[r, pl.ds(0, 16)] = zro
            return 0
        lax.fori_loop(0, R8, zrow, 0)

        row_lo, n8 = _row_span(s)

        def zacc(r, _):
            pltpu.sync_copy(zbd, accd.at[pl.ds(row_lo + r * R8, R8)])
            return 0
        lax.fori_loop(0, n8, zacc, 0)
        plsc.subcore_barrier()

        pltpu.sync_copy(dst_h.at[tid], dstall)
        pltpu.sync_copy(w_h.at[pl.ds(tid * E_TILE, E_TILE)], wall)

        def chunk_body(i, _):

            def group_body(g, _):
                exvec = wall[pl.ds(i * CHUNK + g * 16, 16)]

                def p2(e16, _):
                    e = g * 16 + e16
                    w = exvec.at[jnp.full((16,), e16, jnp.int32)].get(
                        mode="promise_in_bounds")
                    den[e, pl.ds(0, 16)] = jnp.where(lane_i == e16, w, 0.0)
                    return 0
                lax.fori_loop(0, 16, p2, 0)
                return 0
            lax.fori_loop(0, NG, group_body, 0)

            pltpu.sync_copy(den, accd.at[dstall.at[i]], add=True)
            return 0
        lax.fori_loop(0, NCHUNK, chunk_body, 0)

        plsc.subcore_barrier()

        def wout(r, _):
            off = row_lo + r * R8
            pltpu.sync_copy(accd.at[pl.ds(off, R8)],
                            out_d.at[c, pl.ds(off, R8)])
            return 0
        lax.fori_loop(0, n8, wout, 0)

    return body(w_edge, dst)


def _sc_edge_attention(xl, xr, att, src, dst):
    num, w_edge = _sc_numerator(xl, xr, att, src, dst)
    den = _sc_denominator(w_edge, dst.reshape(NW, NCHUNK, CHUNK))
    return num, den


def _tc_first_kernel(x_ref, wl_ref, wr_ref, xl_ref, xr_ref):
    x = x_ref[...]
    xl_ref[...] = jnp.dot(x, wl_ref[...], preferred_element_type=jnp.float32)
    xr_ref[...] = jnp.dot(x, wr_ref[...], preferred_element_type=jnp.float32)


def _tc_first(x, wl, wr):
    return pl.pallas_call(
        _tc_first_kernel,
        grid=(NGRID,),
        in_specs=[
            pl.BlockSpec((NBLK, DIM), lambda i: (i, 0)),
            pl.BlockSpec((DIM, DIM), lambda i: (0, 0)),
            pl.BlockSpec((DIM, DIM), lambda i: (0, 0)),
        ],
        out_specs=[
            pl.BlockSpec((NBLK, DIM), lambda i: (i, 0)),
            pl.BlockSpec((NBLK, DIM), lambda i: (i, 0)),
        ],
        out_shape=[
            jax.ShapeDtypeStruct((N, DIM), jnp.float32),
            jax.ShapeDtypeStruct((N, DIM), jnp.float32),
        ],
    )(x, wl, wr)


def _tc_mid_kernel(n_ref, d_ref, b_ref, wl_ref, wr_ref, xl_ref, xr_ref):
    num = n_ref[0] + n_ref[1]
    den = jnp.sum(d_ref[0] + d_ref[1], axis=1, keepdims=True)
    h = jnp.maximum(num / (den + 1e-16) + b_ref[0], 0.0)
    xl_ref[...] = jnp.dot(h, wl_ref[...], preferred_element_type=jnp.float32)
    xr_ref[...] = jnp.dot(h, wr_ref[...], preferred_element_type=jnp.float32)


def _tc_mid(num, den, b, wl, wr):
    return pl.pallas_call(
        _tc_mid_kernel,
        grid=(NGRID,),
        in_specs=[
            pl.BlockSpec((NC, NBLK, DIM), lambda i: (0, i, 0)),
            pl.BlockSpec((NC, NBLK, 16), lambda i: (0, i, 0)),
            pl.BlockSpec((1, DIM), lambda i: (0, 0)),
            pl.BlockSpec((DIM, DIM), lambda i: (0, 0)),
            pl.BlockSpec((DIM, DIM), lambda i: (0, 0)),
        ],
        out_specs=[
            pl.BlockSpec((NBLK, DIM), lambda i: (i, 0)),
            pl.BlockSpec((NBLK, DIM), lambda i: (i, 0)),
        ],
        out_shape=[
            jax.ShapeDtypeStruct((N, DIM), jnp.float32),
            jax.ShapeDtypeStruct((N, DIM), jnp.float32),
        ],
    )(num, den, b.reshape(1, DIM), wl, wr)


def _tc_final_kernel(n_ref, d_ref, b_ref, batch_ref, wout_ref, bout_ref, out_ref):
    i = pl.program_id(0)
    num = n_ref[0] + n_ref[1]
    den = jnp.sum(d_ref[0] + d_ref[1], axis=1, keepdims=True)
    h = jnp.maximum(num / (den + 1e-16) + b_ref[0], 0.0)
    v = jnp.dot(h, wout_ref[...], preferred_element_type=jnp.float32)
    bb = batch_ref[0, 0, :]
    gids = lax.broadcasted_iota(jnp.int32, (NUM_GRAPHS, NBLK), 0)
    oh = (bb[None, :] == gids).astype(jnp.float32)
    contrib = jnp.dot(oh, v, preferred_element_type=jnp.float32)

    @pl.when(i == 0)
    def _():
        out_ref[...] = contrib + bout_ref[0, 0]

    @pl.when(i > 0)
    def _():
        out_ref[...] = out_ref[...] + contrib


def _tc_final(num, den, b, batch3, wout, bout):
    return pl.pallas_call(
        _tc_final_kernel,
        grid=(NGRID,),
        in_specs=[
            pl.BlockSpec((NC, NBLK, DIM), lambda i: (0, i, 0)),
            pl.BlockSpec((NC, NBLK, 16), lambda i: (0, i, 0)),
            pl.BlockSpec((1, DIM), lambda i: (0, 0)),
            pl.BlockSpec((1, 1, NBLK), lambda i: (i, 0, 0)),
            pl.BlockSpec((DIM, 1), lambda i: (0, 0)),
            pl.BlockSpec((1, 1), lambda i: (0, 0)),
        ],
        out_specs=pl.BlockSpec((NUM_GRAPHS, 1), lambda i: (0, 0)),
        out_shape=jax.ShapeDtypeStruct((NUM_GRAPHS, 1), jnp.float32),
    )(num, den, b.reshape(1, DIM), batch3, wout, bout.reshape(1, 1))


def kernel(x, edge_index, batch, Wl1, Wr1, att1, b1, Wln, Wrn, attn, bn, Wout, bout):
    src = edge_index[0].astype(jnp.int32)
    dst = edge_index[1].astype(jnp.int32)
    batch3 = batch.astype(jnp.int32).reshape(NGRID, 1, NBLK)

    xl, xr = _tc_first(x, Wl1, Wr1)
    num, den = _sc_edge_attention(xl, xr, att1, src, dst)
    xl, xr = _tc_mid(num, den, b1, Wln, Wrn)
    num, den = _sc_edge_attention(xl, xr, attn, src, dst)
    xl, xr = _tc_mid(num, den, bn, Wln, Wrn)
    num, den = _sc_edge_attention(xl, xr, attn, src, dst)
    return _tc_final(num, den, bn, batch3, Wout, bout)


# speedup vs baseline: 8.5015x; 1.0500x over previous
import functools

import jax
import jax.numpy as jnp
from jax import lax
from jax.experimental import pallas as pl
from jax.experimental.pallas import tpu as pltpu
from jax.experimental.pallas import tpu_sc as plsc

N = 10000
E = 320000
DIM = 128
NUM_GRAPHS = 256

NC = 2
NS = 16
NW = NC * NS
E_TILE = E // NW
CHUNK = 80
NG = CHUNK // 16
NCHUNK = E_TILE // CHUNK
ROW_TILE = 624
R8 = 8

NBLK = 1000
NGRID = N // NBLK


def _row_span(s):
    row_lo = s * ROW_TILE
    n8 = jnp.where(s == NS - 1, (N - (NS - 1) * ROW_TILE) // R8, ROW_TILE // R8)
    return row_lo, n8


def _sc_numerator(xl, xr, att, sd4):
    mesh = plsc.VectorSubcoreMesh(core_axis_name="c", subcore_axis_name="s")

    @functools.partial(
        pl.kernel,
        out_type=(
            jax.ShapeDtypeStruct((NC, N, DIM), jnp.float32),
            jax.ShapeDtypeStruct((E,), jnp.float32),
        ),
        mesh=mesh,
        scratch_types=(
            pltpu.VMEM_SHARED((N, DIM), jnp.float32),
            pltpu.VMEM((2, CHUNK), jnp.int32),
            pltpu.VMEM((CHUNK, DIM), jnp.float32),
            pltpu.VMEM((CHUNK, DIM), jnp.float32),
            pltpu.VMEM((CHUNK,), jnp.float32),
            pltpu.VMEM((DIM,), jnp.float32),
            pltpu.VMEM((R8, DIM), jnp.float32),
        ),
    )
    def body(xl_h, xr_h, att_h, sd_h, out_n, out_w,
             accn, sdv, xlr, xrr, wv, attv, zbn):
        c = lax.axis_index("c")
        s = lax.axis_index("s")
        tid = c * NS + s
        zro = jnp.zeros((16,), jnp.float32)

        def zrow(r, _):
            for k in range(DIM // 16):
                zbn[r, pl.ds(k * 16, 16)] = zro
            return 0
        lax.fori_loop(0, R8, zrow, 0)

        row_lo, n8 = _row_span(s)

        def zacc(r, _):
            pltpu.sync_copy(zbn, accn.at[pl.ds(row_lo + r * R8, R8)])
            return 0
        lax.fori_loop(0, n8, zacc, 0)
        plsc.subcore_barrier()

        pltpu.sync_copy(att_h, attv)
        attc = [attv[pl.ds(k * 16, 16)] for k in range(DIM // 16)]
        lane_i = lax.broadcasted_iota(jnp.int32, (16,), 0)
        perms = [jnp.bitwise_xor(lane_i, sh) for sh in (1, 2, 4, 8)]

        def hsum(v):
            for pm in perms:
                v = v + v.at[pm].get(mode="promise_in_bounds")
            return v

        def chunk_body(i, _):
            base = tid * E_TILE + i * CHUNK
            pltpu.sync_copy(sd_h.at[tid, i], sdv)
            pltpu.sync_copy(xl_h.at[sdv.at[0]], xlr)
            pltpu.sync_copy(xr_h.at[sdv.at[1]], xrr)

            def group_body(g, _):
                def p1(e16, evec):
                    e = g * 16 + e16
                    p = zro
                    for k in range(DIM // 16):
                        sl = pl.ds(k * 16, 16)
                        z = xlr[e, sl] + xrr[e, sl]
                        y = jnp.maximum(z, 0.2 * z)
                        p = p + y * attc[k]
                    return jnp.where(lane_i == e16, hsum(p), evec)
                evec = lax.fori_loop(0, 16, p1, zro)
                exvec = jnp.exp(evec)
                wv[pl.ds(g * 16, 16)] = exvec

                def p2(e16, _):
                    e = g * 16 + e16
                    w = exvec.at[jnp.full((16,), e16, jnp.int32)].get(
                        mode="promise_in_bounds")
                    for k in range(DIM // 16):
                        sl = pl.ds(k * 16, 16)
                        xlr[e, sl] = w * xlr[e, sl]
                    return 0
                lax.fori_loop(0, 16, p2, 0)
                return 0
            lax.fori_loop(0, NG, group_body, 0)

            pltpu.sync_copy(wv, out_w.at[pl.ds(base, CHUNK)])
            pltpu.sync_copy(xlr, accn.at[sdv.at[1]], add=True)
            return 0
        lax.fori_loop(0, NCHUNK, chunk_body, 0)

        plsc.subcore_barrier()

        def wout(r, _):
            off = row_lo + r * R8
            pltpu.sync_copy(accn.at[pl.ds(off, R8)],
                            out_n.at[c, pl.ds(off, R8)])
            return 0
        lax.fori_loop(0, n8, wout, 0)

    return body(xl, xr, att, sd4)


def _sc_denominator(w_edge, dst):
    mesh = plsc.VectorSubcoreMesh(core_axis_name="c", subcore_axis_name="s")

    @functools.partial(
        pl.kernel,
        out_type=jax.ShapeDtypeStruct((NC, N, 16), jnp.float32),
        mesh=mesh,
        scratch_types=(
            pltpu.VMEM_SHARED((N, 16), jnp.float32),
            pltpu.VMEM((NCHUNK, CHUNK), jnp.int32),
            pltpu.VMEM((E_TILE,), jnp.float32),
            pltpu.VMEM((CHUNK, 16), jnp.float32),
            pltpu.VMEM((R8, 16), jnp.float32),
        ),
    )
    def body(w_h, dst_h, out_d, accd, dstall, wall, den, zbd):
        c = lax.axis_index("c")
        s = lax.axis_index("s")
        tid = c * NS + s
        zro = jnp.zeros((16,), jnp.float32)
        lane_i = lax.broadcasted_iota(jnp.int32, (16,), 0)

        def zrow(r, _):
            zbd[r, pl.ds(0, 16)] = zro
            return 0
        lax.fori_loop(0, R8, zrow, 0)

        row_lo, n8 = _row_span(s)

        def zacc(r, _):
            pltpu.sync_copy(zbd, accd.at[pl.ds(row_lo + r * R8, R8)])
            return 0
        lax.fori_loop(0, n8, zacc, 0)
        plsc.subcore_barrier()

        pltpu.sync_copy(dst_h.at[tid], dstall)
        pltpu.sync_copy(w_h.at[pl.ds(tid * E_TILE, E_TILE)], wall)

        def chunk_body(i, _):

            def group_body(g, _):
                exvec = wall[pl.ds(i * CHUNK + g * 16, 16)]

                def p2(e16, _):
                    e = g * 16 + e16
                    w = exvec.at[jnp.full((16,), e16, jnp.int32)].get(
                        mode="promise_in_bounds")
                    den[e, pl.ds(0, 16)] = jnp.where(lane_i == e16, w, 0.0)
                    return 0
                lax.fori_loop(0, 16, p2, 0)
                return 0
            lax.fori_loop(0, NG, group_body, 0)

            pltpu.sync_copy(den, accd.at[dstall.at[i]], add=True)
            return 0
        lax.fori_loop(0, NCHUNK, chunk_body, 0)

        plsc.subcore_barrier()

        def wout(r, _):
            off = row_lo + r * R8
            pltpu.sync_copy(accd.at[pl.ds(off, R8)],
                            out_d.at[c, pl.ds(off, R8)])
            return 0
        lax.fori_loop(0, n8, wout, 0)

    return body(w_edge, dst)


def _sc_edge_attention(xl, xr, att, src, dst):
    src3 = src.reshape(NW, NCHUNK, CHUNK)
    dst3 = dst.reshape(NW, NCHUNK, CHUNK)
    sd4 = jnp.stack([src3, dst3], axis=2)
    num, w_edge = _sc_numerator(xl, xr, att, sd4)
    den = _sc_denominator(w_edge, dst3)
    return num, den


def _tc_first_kernel(x_ref, wl_ref, wr_ref, xl_ref, xr_ref):
    x = x_ref[...]
    xl_ref[...] = jnp.dot(x, wl_ref[...], preferred_element_type=jnp.float32)
    xr_ref[...] = jnp.dot(x, wr_ref[...], preferred_element_type=jnp.float32)


def _tc_first(x, wl, wr):
    return pl.pallas_call(
        _tc_first_kernel,
        grid=(NGRID,),
        in_specs=[
            pl.BlockSpec((NBLK, DIM), lambda i: (i, 0)),
            pl.BlockSpec((DIM, DIM), lambda i: (0, 0)),
            pl.BlockSpec((DIM, DIM), lambda i: (0, 0)),
        ],
        out_specs=[
            pl.BlockSpec((NBLK, DIM), lambda i: (i, 0)),
            pl.BlockSpec((NBLK, DIM), lambda i: (i, 0)),
        ],
        out_shape=[
            jax.ShapeDtypeStruct((N, DIM), jnp.float32),
            jax.ShapeDtypeStruct((N, DIM), jnp.float32),
        ],
    )(x, wl, wr)


def _tc_mid_kernel(n_ref, d_ref, b_ref, wl_ref, wr_ref, xl_ref, xr_ref):
    num = n_ref[0] + n_ref[1]
    den = jnp.sum(d_ref[0] + d_ref[1], axis=1, keepdims=True)
    h = jnp.maximum(num / (den + 1e-16) + b_ref[0], 0.0)
    xl_ref[...] = jnp.dot(h, wl_ref[...], preferred_element_type=jnp.float32)
    xr_ref[...] = jnp.dot(h, wr_ref[...], preferred_element_type=jnp.float32)


def _tc_mid(num, den, b, wl, wr):
    return pl.pallas_call(
        _tc_mid_kernel,
        grid=(NGRID,),
        in_specs=[
            pl.BlockSpec((NC, NBLK, DIM), lambda i: (0, i, 0)),
            pl.BlockSpec((NC, NBLK, 16), lambda i: (0, i, 0)),
            pl.BlockSpec((1, DIM), lambda i: (0, 0)),
            pl.BlockSpec((DIM, DIM), lambda i: (0, 0)),
            pl.BlockSpec((DIM, DIM), lambda i: (0, 0)),
        ],
        out_specs=[
            pl.BlockSpec((NBLK, DIM), lambda i: (i, 0)),
            pl.BlockSpec((NBLK, DIM), lambda i: (i, 0)),
        ],
        out_shape=[
            jax.ShapeDtypeStruct((N, DIM), jnp.float32),
            jax.ShapeDtypeStruct((N, DIM), jnp.float32),
        ],
    )(num, den, b.reshape(1, DIM), wl, wr)


def _tc_final_kernel(n_ref, d_ref, b_ref, batch_ref, wout_ref, bout_ref, out_ref):
    i = pl.program_id(0)
    num = n_ref[0] + n_ref[1]
    den = jnp.sum(d_ref[0] + d_ref[1], axis=1, keepdims=True)
    h = jnp.maximum(num / (den + 1e-16) + b_ref[0], 0.0)
    v = jnp.dot(h, wout_ref[...], preferred_element_type=jnp.float32)
    bb = batch_ref[0, 0, :]
    gids = lax.broadcasted_iota(jnp.int32, (NUM_GRAPHS, NBLK), 0)
    oh = (bb[None, :] == gids).astype(jnp.float32)
    contrib = jnp.dot(oh, v, preferred_element_type=jnp.float32)

    @pl.when(i == 0)
    def _():
        out_ref[...] = contrib + bout_ref[0, 0]

    @pl.when(i > 0)
    def _():
        out_ref[...] = out_ref[...] + contrib


def _tc_final(num, den, b, batch3, wout, bout):
    return pl.pallas_call(
        _tc_final_kernel,
        grid=(NGRID,),
        in_specs=[
            pl.BlockSpec((NC, NBLK, DIM), lambda i: (0, i, 0)),
            pl.BlockSpec((NC, NBLK, 16), lambda i: (0, i, 0)),
            pl.BlockSpec((1, DIM), lambda i: (0, 0)),
            pl.BlockSpec((1, 1, NBLK), lambda i: (i, 0, 0)),
            pl.BlockSpec((DIM, 1), lambda i: (0, 0)),
            pl.BlockSpec((1, 1), lambda i: (0, 0)),
        ],
        out_specs=pl.BlockSpec((NUM_GRAPHS, 1), lambda i: (0, 0)),
        out_shape=jax.ShapeDtypeStruct((NUM_GRAPHS, 1), jnp.float32),
    )(num, den, b.reshape(1, DIM), batch3, wout, bout.reshape(1, 1))


def kernel(x, edge_index, batch, Wl1, Wr1, att1, b1, Wln, Wrn, attn, bn, Wout, bout):
    src = edge_index[0].astype(jnp.int32)
    dst = edge_index[1].astype(jnp.int32)
    batch3 = batch.astype(jnp.int32).reshape(NGRID, 1, NBLK)

    xl, xr = _tc_first(x, Wl1, Wr1)
    num, den = _sc_edge_attention(xl, xr, att1, src, dst)
    xl, xr = _tc_mid(num, den, b1, Wln, Wrn)
    num, den = _sc_edge_attention(xl, xr, attn, src, dst)
    xl, xr = _tc_mid(num, den, bn, Wln, Wrn)
    num, den = _sc_edge_attention(xl, xr, attn, src, dst)
    return _tc_final(num, den, bn, batch3, Wout, bout)
